# Initial kernel scaffold; baseline (speedup 1.0000x reference)
import jax
import jax.numpy as jnp
from jax.experimental import pallas as pl


def kernel(x, org_node_pos, org_node_size, b_shape, b_iou, eps, params, edge_index, batch, ptr):
    raise NotImplementedError("write your pallas kernel here")


# speedup vs baseline: 1.0003x; 1.0003x over previous
import jax
import jax.numpy as jnp
from jax.experimental import pallas as pl


def _identity_kernel(x_ref, o_ref):
    o_ref[...] = x_ref[...]


def _linear(x, p):
    return x @ p["w"] + p["b"]


def _gcn(x, src, dst, p, n):
    xw = x @ p["w"]
    deg = jnp.zeros((n,), x.dtype).at[dst].add(1.0)
    dinv = jnp.where(deg > 0, 1.0 / jnp.sqrt(deg), 0.0)
    norm = dinv[src] * dinv[dst]
    out = jnp.zeros((n, xw.shape[1]), x.dtype).at[dst].add(xw[src] * norm[:, None])
    return out + p["b"]


def _mean_pool(x, batch, nb):
    s = jax.ops.segment_sum(x, batch, num_segments=nb)
    c = jax.ops.segment_sum(jnp.ones((x.shape[0],), x.dtype), batch, num_segments=nb)
    return s / jnp.maximum(c, 1.0)[:, None]


def kernel(x, org_node_pos, org_node_size, b_shape, b_iou, eps, params, edge_index, batch, ptr):
    nb = ptr.shape[0] - 1
    n = x.shape[0]
    relu = jax.nn.relu
    loop = jnp.arange(n)
    src = jnp.concatenate([edge_index[0], loop])
    dst = jnp.concatenate([edge_index[1], loop])
    bs = _linear(b_shape, params["enc_shape"])
    bi = _linear(b_iou, params["enc_iou"])
    shape_feature = jnp.concatenate([bs, bi], 1)
    pos = relu(_linear(org_node_pos, params["pos_init"]))
    size = relu(_linear(org_node_size, params["size_init"]))
    xe = _linear(x, params["ex_init"])
    one_hot = jnp.tile(jnp.eye(80, dtype=x.dtype), (nb, 1))[:n]
    xe = jnp.concatenate([xe, one_hot], 1)
    cond_emb = jnp.zeros((n, 128), x.dtype)
    xe = jnp.concatenate([xe, cond_emb], 1)
    ft = relu(_linear(xe, params["ft_init"]))
    n0 = jnp.concatenate([shape_feature, size, pos, ft], 1)
    n0 = pl.pallas_call(
        _identity_kernel,
        out_shape=jax.ShapeDtypeStruct(n0.shape, n0.dtype),
    )(n0)
    n1 = relu(_gcn(n0, src, dst, params["e_conv1"], n))
    n2 = relu(_gcn(n1, src, dst, params["e_conv2"], n))
    n3 = relu(_gcn(n2, src, dst, params["e_conv3"], n))
    g = jnp.concatenate([_mean_pool(n0, batch, nb), _mean_pool(n1, batch, nb), _mean_pool(n2, batch, nb), _mean_pool(n3, batch, nb)], 1)
    zhid = _linear(g, params["aggregate"])
    mu = _linear(zhid, params["fc_mu"])
    log_var = _linear(zhid, params["fc_var"])
    z = eps * jnp.exp(0.5 * log_var) + mu
    zd = _linear(z, params["d_ft_init"]).reshape(nb * 80, 256)[:n]
    zd = jnp.concatenate([zd, one_hot], 1)
    d1 = relu(_gcn(zd, src, dst, params["d_conv1"], n))
    d2 = relu(_gcn(d1, src, dst, params["d_conv2"], n))
    d3 = relu(_gcn(d2, src, dst, params["d_conv3"], n))
    def head(p0, p1):
        return _linear(relu(_linear(d3, params[p0])), params[p1])
    exist = head("d_exist_0", "d_exist_1")
    posx = head("d_posx_0", "d_posx_1")
    posy = head("d_posy_0", "d_posy_1")
    sizex = head("d_sizex_0", "d_sizex_1")
    sizey = head("d_sizey_0", "d_sizey_1")
    bso = head("d_shape_0", "d_shape_1")
    bio = head("d_iou_0", "d_iou_1")
    return (exist, posx, posy, sizex, sizey, bso, bio, mu, log_var)


# speedup vs baseline: 2.9163x; 2.9153x over previous
import functools

import jax
import jax.numpy as jnp
from jax import lax
from jax.experimental import pallas as pl
from jax.experimental.pallas import tpu as pltpu
from jax.experimental.pallas import tpu_sc as plsc

NN = 10000
NB = 125
NPG = 80
CH = 256
E_TOT = 170000
NSUB = 16
NCORE = 2
EPW = 10752
E_PAD = EPW * NSUB
CHUNK = 128
NCHUNK = EPW // CHUNK
HALF = NN // 2
TRASH = HALF
ACC_ROWS = 5120
RB = 400
NBLK = NN // RB
def _relu(v):
    return jnp.maximum(v, 0.0)


def _dot(a, b):
    return jnp.dot(a.astype(jnp.bfloat16), b.astype(jnp.bfloat16),
                   preferred_element_type=jnp.float32)


def _b16(w):
    return w.astype(jnp.bfloat16).astype(jnp.float32)


def _sc_mesh():
    return plsc.VectorSubcoreMesh(core_axis_name="c", subcore_axis_name="s",
                                  num_cores=NCORE, num_subcores=NSUB)


def _sc_agg_body(y0_hbm, y1_hbm, src_hbm, ldst_hbm, zeros_hbm,
                 a0_hbm, a1_hbm, src_v, ldst_v, rows_v, rows1_v,
                 acc0, acc1, sem):
    c = lax.axis_index("c")
    s = lax.axis_index("s")
    pltpu.sync_copy(zeros_hbm, rows_v)
    for acc in (acc0, acc1):
        pltpu.sync_copy(rows_v, acc.at[pl.ds(s * 320, CHUNK)])
        pltpu.sync_copy(rows_v, acc.at[pl.ds(s * 320 + CHUNK, CHUNK)])
        pltpu.sync_copy(rows_v.at[pl.ds(0, 64)],
                        acc.at[pl.ds(s * 320 + 2 * CHUNK, 64)])
    plsc.subcore_barrier()
    ebase = s * EPW
    lbase = c * E_PAD + s * EPW

    def body(k, carry):
        off = k * CHUNK
        pltpu.sync_copy(src_hbm.at[pl.ds(ebase + off, CHUNK)], src_v)
        pltpu.sync_copy(ldst_hbm.at[pl.ds(lbase + off, CHUNK)], ldst_v)
        pltpu.async_copy(y0_hbm.at[src_v], rows_v, sem).wait()
        pltpu.sync_copy(rows_v, acc0.at[ldst_v], add=True)
        pltpu.async_copy(y1_hbm.at[src_v], rows1_v, sem).wait()
        pltpu.sync_copy(rows1_v, acc1.at[ldst_v], add=True)
        return carry

    lax.fori_loop(0, NCHUNK, body, 0)
    plsc.subcore_barrier()
    base = c * HALF

    for acc, out in ((acc0, a0_hbm), (acc1, a1_hbm)):
        @pl.when(s < NSUB - 1)
        def _():
            for off, n in ((0, CHUNK), (CHUNK, CHUNK), (2 * CHUNK, 64)):
                pltpu.sync_copy(acc.at[pl.ds(s * 320 + off, n)],
                                rows_v.at[pl.ds(0, n)])
                pltpu.sync_copy(rows_v.at[pl.ds(0, n)],
                                out.at[pl.ds(base + s * 320 + off, n)])

        @pl.when(s == NSUB - 1)
        def _():
            for off, n in ((0, CHUNK), (CHUNK, 72)):
                pltpu.sync_copy(acc.at[pl.ds((NSUB - 1) * 320 + off, n)],
                                rows_v.at[pl.ds(0, n)])
                pltpu.sync_copy(rows_v.at[pl.ds(0, n)],
                                out.at[pl.ds(base + (NSUB - 1) * 320 + off, n)])


@functools.cache
def _sc_kernels():
    agg = functools.partial(
        pl.kernel,
        out_type=[jax.ShapeDtypeStruct((NN, 128), jnp.float32),
                  jax.ShapeDtypeStruct((NN, 128), jnp.float32)],
        mesh=_sc_mesh(),
        scratch_types=[
            pltpu.VMEM((CHUNK,), jnp.int32),
            pltpu.VMEM((CHUNK,), jnp.int32),
            pltpu.VMEM((CHUNK, 128), jnp.float32),
            pltpu.VMEM((CHUNK, 128), jnp.float32),
            pltpu.VMEM_SHARED((ACC_ROWS, 128), jnp.float32),
            pltpu.VMEM_SHARED((ACC_ROWS, 128), jnp.float32),
            pltpu.SemaphoreType.DMA,
        ],
    )(_sc_agg_body)
    return agg


def _sc_agg(*args):
    return _sc_kernels()(*args)


def _dinv_of(dv_blk):
    return dv_blk[:, 0:1]


def _pool_mat():
    r5 = lax.broadcasted_iota(jnp.int32, (5, RB), 0)
    rr = lax.broadcasted_iota(jnp.int32, (5, RB), 1)
    return jnp.where(r5 == rr // NPG, 1.0, 0.0)


def _pool(h):
    s = jnp.dot(_pool_mat(), h, preferred_element_type=jnp.float32,
                precision=lax.Precision.HIGHEST)
    return s / float(NPG)


def _enc_body(bsh, biou, osz, opo, x, deg,
              wes, bes, wei, bei, ws, bs_, wp, bp_, wex, bex,
              wft1, woh, bft, we1, y1a, y1b, p0):
    sf = _dot(bsh[...], wes[...]) + bes[...]
    bi = _dot(biou[...], wei[...]) + bei[...]
    size = _relu(_dot(osz[...], ws[...]) + bs_[...])
    pos = _relu(_dot(opo[...], wp[...]) + bp_[...])
    xe = _dot(x[...], wex[...]) + bex[...]
    oh5 = jnp.concatenate([_b16(woh[...])] * 5, axis=0)
    ft = _relu(_dot(xe, wft1[...]) + oh5 + bft[...])
    n0 = jnp.concatenate([sf, bi, size, pos, ft], axis=1)
    dinv = _dinv_of(deg[...])
    yl = dinv * _dot(n0, we1[...])
    y1a[...] = yl[:, :128]
    y1b[...] = yl[:, 128:]
    p0[...] = _pool(n0)[None]


def _enc_call(bsh, biou, osz, opo, x, deg, wes, bes, wei, bei, ws, bs_, wp,
              bp_, wex, bex, wft1, woh, bft, we1):
    full = lambda shape: pl.BlockSpec(shape, lambda i: (0, 0))
    row = lambda w: pl.BlockSpec((RB, w), lambda i: (i, 0))
    return pl.pallas_call(
        _enc_body,
        grid=(NBLK,),
        in_specs=[row(6), row(1), row(2), row(2), row(2), row(1),
                  full((6, 64)), full((1, 64)), full((1, 64)), full((1, 64)),
                  full((2, 128)), full((1, 128)), full((2, 128)), full((1, 128)),
                  full((2, 64)), full((1, 64)),
                  full((64, 128)), full((NPG, 128)), full((1, 128)),
                  full((512, 256))],
        out_specs=[pl.BlockSpec((RB, 128), lambda i: (i, 0)),
                   pl.BlockSpec((RB, 128), lambda i: (i, 0)),
                   pl.BlockSpec((1, 5, 512), lambda i: (i, 0, 0))],
        out_shape=[jax.ShapeDtypeStruct((NN, 128), jnp.float32),
                   jax.ShapeDtypeStruct((NN, 128), jnp.float32),
                   jax.ShapeDtypeStruct((NBLK, 5, 512), jnp.float32)],
    )(bsh, biou, osz, opo, x, deg, wes, bes, wei, bei, ws, bs_, wp, bp_,
      wex, bex, wft1, woh, bft, we1)


def _conv_body_full(a0, a1, deg, w, b, y0, y1, p):
    dinv = _dinv_of(deg[...])
    h = _relu(dinv * jnp.concatenate([a0[...], a1[...]], axis=1) + b[...])
    yl = dinv * _dot(h, w[...])
    y0[...] = yl[:, :128]
    y1[...] = yl[:, 128:]
    p[...] = _pool(h)[None]


def _conv_body_y(a0, a1, deg, w, b, y0, y1):
    dinv = _dinv_of(deg[...])
    h = _relu(dinv * jnp.concatenate([a0[...], a1[...]], axis=1) + b[...])
    yl = dinv * _dot(h, w[...])
    y0[...] = yl[:, :128]
    y1[...] = yl[:, 128:]


def _conv_body_p(a0, a1, deg, b, p):
    dinv = _dinv_of(deg[...])
    h = _relu(dinv * jnp.concatenate([a0[...], a1[...]], axis=1) + b[...])
    p[...] = _pool(h)[None]


def _conv_call(acc, deg, w, b, want_y, want_p):
    a0, a1 = acc
    full = lambda shape: pl.BlockSpec(shape, lambda i: (0, 0))
    row = lambda wd: pl.BlockSpec((RB, wd), lambda i: (i, 0))
    p_spec = pl.BlockSpec((1, 5, 256), lambda i: (i, 0, 0))
    y_sh = jax.ShapeDtypeStruct((NN, 128), jnp.float32)
    p_sh = jax.ShapeDtypeStruct((NBLK, 5, 256), jnp.float32)
    if want_y and want_p:
        return pl.pallas_call(
            _conv_body_full, grid=(NBLK,),
            in_specs=[row(128), row(128), row(1), full((256, 256)),
                      full((1, 256))],
            out_specs=[row(128), row(128), p_spec],
            out_shape=[y_sh, y_sh, p_sh],
        )(a0, a1, deg, w, b)
    if want_y:
        return pl.pallas_call(
            _conv_body_y, grid=(NBLK,),
            in_specs=[row(128), row(128), row(1), full((256, 256)),
                      full((1, 256))],
            out_specs=[row(128), row(128)], out_shape=[y_sh, y_sh],
        )(a0, a1, deg, w, b)
    return pl.pallas_call(
        _conv_body_p, grid=(NBLK,),
        in_specs=[row(128), row(128), row(1), full((1, 256))],
        out_specs=p_spec, out_shape=p_sh,
    )(a0, a1, deg, b)


def _latent_body(p0, p1, p2, p3, eps, wagg, bagg, wmu, bmu, wvar, bvar,
                 mu, logv, z):
    g = jnp.concatenate([p0[...], p1[...], p2[...], p3[...]], axis=1)
    zhid = _dot(g, wagg[...]) + bagg[...]
    m = _dot(zhid, wmu[...]) + bmu[...]
    lv = _dot(zhid, wvar[...]) + bvar[...]
    mu[...] = m
    logv[...] = lv
    z[...] = eps[...] * jnp.exp(0.5 * lv) + m


def _latent_call(p0, p1, p2, p3, eps, wagg, bagg, wmu, bmu, wvar, bvar):
    full = lambda shape: pl.BlockSpec(shape, lambda: (0, 0))
    sh = lambda s: jax.ShapeDtypeStruct(s, jnp.float32)
    return pl.pallas_call(
        _latent_body,
        in_specs=[full((NB, 512)), full((NB, 256)), full((NB, 256)),
                  full((NB, 256)), full((NB, 256)), full((1280, 256)),
                  full((1, 256)), full((256, 256)), full((1, 256)),
                  full((256, 256)), full((1, 256))],
        out_specs=[full((NB, 256)), full((NB, 256)), full((NB, 256))],
        out_shape=[sh((NB, 256)), sh((NB, 256)), sh((NB, 256))],
    )(p0, p1, p2, p3, eps, wagg, bagg, wmu, bmu, wvar, bvar)


def _dft_body(z, w, b, zw):
    zw[...] = _dot(z[...], w[...]) + b[...]


def _dft_call(z, wdft, bdft):
    return pl.pallas_call(
        _dft_body,
        grid=(20,),
        in_specs=[pl.BlockSpec((NB, 256), lambda i: (0, 0)),
                  pl.BlockSpec((256, 1024), lambda i: (0, i)),
                  pl.BlockSpec((1, 1024), lambda i: (0, i))],
        out_specs=pl.BlockSpec((NB, 1024), lambda i: (0, i)),
        out_shape=jax.ShapeDtypeStruct((NB, 20480), jnp.float32),
    )(z, wdft, bdft)


def _dec1_body(zd, deg, w1a, w1b, y0, y1):
    dinv = _dinv_of(deg[...])
    oh5 = jnp.concatenate([_b16(w1b[...])] * 5, axis=0)
    yl = dinv * (_dot(zd[...], w1a[...]) + oh5)
    y0[...] = yl[:, :128]
    y1[...] = yl[:, 128:]


def _dec1_call(zd, deg, w1a, w1b):
    full = lambda shape: pl.BlockSpec(shape, lambda i: (0, 0))
    return pl.pallas_call(
        _dec1_body,
        grid=(NBLK,),
        in_specs=[pl.BlockSpec((RB, 256), lambda i: (i, 0)),
                  pl.BlockSpec((RB, 1), lambda i: (i, 0)),
                  full((256, 256)), full((NPG, 256))],
        out_specs=[pl.BlockSpec((RB, 128), lambda i: (i, 0)),
                   pl.BlockSpec((RB, 128), lambda i: (i, 0))],
        out_shape=[jax.ShapeDtypeStruct((NN, 128), jnp.float32),
                   jax.ShapeDtypeStruct((NN, 128), jnp.float32)],
    )(zd, deg, w1a, w1b)


def _heads_body(a0, a1, deg, b3, w0, b0, w1, b1, out):
    dinv = _dinv_of(deg[...])
    d3 = _relu(dinv * jnp.concatenate([a0[...], a1[...]], axis=1) + b3[...])
    h = _relu(_dot(d3, w0[...]) + b0[...])
    out[...] = _dot(h, w1[...]) + b1[...]


def _heads_call(acc, deg, b3, w0cat, b0cat, w1bd, b1cat):
    a0, a1 = acc
    full = lambda shape: pl.BlockSpec(shape, lambda i: (0, 0))
    return pl.pallas_call(
        _heads_body,
        grid=(NBLK,),
        in_specs=[pl.BlockSpec((RB, 128), lambda i: (i, 0)),
                  pl.BlockSpec((RB, 128), lambda i: (i, 0)),
                  pl.BlockSpec((RB, 1), lambda i: (i, 0)),
                  full((1, 256)), full((256, 1792)), full((1, 1792)),
                  full((1792, 12)), full((1, 12))],
        out_specs=pl.BlockSpec((RB, 12), lambda i: (i, 0)),
        out_shape=jax.ShapeDtypeStruct((NN, 12), jnp.float32),
    )(a0, a1, deg, b3, w0cat, b0cat, w1bd, b1cat)


_HEADS = ["d_exist", "d_posx", "d_posy", "d_sizex", "d_sizey", "d_shape", "d_iou"]
_HEAD_W = [1, 1, 1, 1, 1, 6, 1]


def kernel(x, org_node_pos, org_node_size, b_shape, b_iou, eps, params,
           edge_index, batch, ptr):
    f32 = jnp.float32
    p = params

    ei = edge_index.astype(jnp.int32)
    loop = jnp.arange(NN, dtype=jnp.int32)
    src = jnp.concatenate([ei[0], loop])
    dst = jnp.concatenate([ei[1], loop])
    pad = E_PAD - E_TOT
    src_p = jnp.concatenate([src, jnp.zeros((pad,), jnp.int32)])
    dst_p = jnp.concatenate([dst, jnp.full((pad,), -1, jnp.int32)])
    l0 = jnp.where((dst_p >= 0) & (dst_p < HALF), dst_p, TRASH)
    l1 = jnp.where(dst_p >= HALF, dst_p - HALF, TRASH)
    ldst = jnp.concatenate([l0, l1])

    zeros128 = jnp.zeros((CHUNK, 128), f32)
    ones_nn = jnp.ones((NN, 128), f32)

    agg = lambda y: _sc_agg(y[0], y[1], src_p, ldst, zeros128)

    degp = agg((ones_nn, ones_nn))[0]
    d0 = degp[:, :1]
    deg = jnp.where(d0 > 0.0, 1.0 / jnp.sqrt(d0), 0.0)

    b2 = lambda name: p[name]["b"].reshape(1, -1)
    y1a, y1b, p0 = _enc_call(
        b_shape, b_iou, org_node_size, org_node_pos, x, deg,
        p["enc_shape"]["w"], b2("enc_shape"), p["enc_iou"]["w"], b2("enc_iou"),
        p["size_init"]["w"], b2("size_init"), p["pos_init"]["w"], b2("pos_init"),
        p["ex_init"]["w"], b2("ex_init"),
        p["ft_init"]["w"][:64], p["ft_init"]["w"][64:144], b2("ft_init"),
        p["e_conv1"]["w"])
    acc1 = agg((y1a, y1b))
    y2a, y2b, p1 = _conv_call(acc1, deg, p["e_conv2"]["w"], b2("e_conv1"),
                              True, True)
    acc2 = agg((y2a, y2b))
    y3a, y3b, p2 = _conv_call(acc2, deg, p["e_conv3"]["w"], b2("e_conv2"),
                              True, True)
    acc3 = agg((y3a, y3b))
    p3 = _conv_call(acc3, deg, None, b2("e_conv3"), False, True)

    p0 = p0.reshape(NB, 512)
    p1 = p1.reshape(NB, 256)
    p2 = p2.reshape(NB, 256)
    p3 = p3.reshape(NB, 256)
    mu, log_var, z = _latent_call(
        p0, p1, p2, p3, eps, p["aggregate"]["w"], b2("aggregate"),
        p["fc_mu"]["w"], b2("fc_mu"), p["fc_var"]["w"], b2("fc_var"))
    zw = _dft_call(z, p["d_ft_init"]["w"], b2("d_ft_init"))
    zd = zw.reshape(NN, CH)

    y_d1 = _dec1_call(zd, deg, p["d_conv1"]["w"][:256], p["d_conv1"]["w"][256:336])
    accd1 = agg(y_d1)
    y_d2 = _conv_call(accd1, deg, p["d_conv2"]["w"], b2("d_conv1"), True, False)
    accd2 = agg(y_d2)
    y_d3 = _conv_call(accd2, deg, p["d_conv3"]["w"], b2("d_conv2"), True, False)
    accd3 = agg(y_d3)

    w0cat = jnp.concatenate([p[h + "_0"]["w"] for h in _HEADS], axis=1)
    b0cat = jnp.concatenate([p[h + "_0"]["b"] for h in _HEADS]).reshape(1, -1)
    cols = []
    for i, h in enumerate(_HEADS):
        w1 = p[h + "_1"]["w"]
        above = sum(_HEAD_W[:i])
        below = sum(_HEAD_W[i + 1:])
        blk = jnp.pad(w1, ((0, 0), (above, below)))
        cols.append(blk)
    w1bd = jnp.concatenate(cols, axis=0)
    b1cat = jnp.concatenate([p[h + "_1"]["b"] for h in _HEADS]).reshape(1, -1)
    out12 = _heads_call(accd3, deg, b2("d_conv3"), w0cat, b0cat, w1bd, b1cat)

    exist = out12[:, 0:1]
    posx = out12[:, 1:2]
    posy = out12[:, 2:3]
    sizex = out12[:, 3:4]
    sizey = out12[:, 4:5]
    bso = out12[:, 5:11]
    bio = out12[:, 11:12]
    return (exist, posx, posy, sizex, sizey, bso, bio, mu, log_var)
